# baseline (speedup 1.0000x reference)
import functools

import jax
import jax.numpy as jnp
from jax import lax
from jax.experimental import pallas as pl
from jax.experimental.pallas import tpu as pltpu
from jax.experimental.pallas import tpu_sc as plsc

_NUM_CORES = 2
_NUM_SUBCORES = 16
_NUM_WORKERS = _NUM_CORES * _NUM_SUBCORES

_BATCH = 16384
_D_MODEL = 768
_ROWS_PER_WORKER = _BATCH // _NUM_WORKERS
_CHUNKS = (160, 160, 160, 32)


@functools.partial(
    pl.kernel,
    out_type=jax.ShapeDtypeStruct((_BATCH, _D_MODEL), jnp.float32),
    mesh=plsc.VectorSubcoreMesh(core_axis_name="c", subcore_axis_name="s"),
    scratch_types=[
        pltpu.VMEM((_ROWS_PER_WORKER,), jnp.int32),
        pltpu.VMEM((160, _D_MODEL), jnp.float32),
        pltpu.SemaphoreType.DMA,
    ],
)
def _sc_gather(idx_hbm, table_hbm, out_hbm, idx_v, rows_v, sem):
    wid = lax.axis_index("s") * _NUM_CORES + lax.axis_index("c")
    base = wid * _ROWS_PER_WORKER
    c0 = _CHUNKS[0]
    pltpu.sync_copy(idx_hbm.at[pl.ds(base, c0)], idx_v.at[pl.ds(0, c0)])
    first = pltpu.async_copy(
        table_hbm.at[idx_v.at[pl.ds(0, c0)]], rows_v.at[pl.ds(0, c0)], sem)
    pltpu.sync_copy(idx_hbm.at[pl.ds(base + c0, _ROWS_PER_WORKER - c0)],
                    idx_v.at[pl.ds(c0, _ROWS_PER_WORKER - c0)])
    first.wait()
    pltpu.sync_copy(rows_v.at[pl.ds(0, c0)], out_hbm.at[pl.ds(base, c0)])

    off = c0
    for c in _CHUNKS[1:]:
        pltpu.async_copy(
            table_hbm.at[idx_v.at[pl.ds(off, c)]],
            rows_v.at[pl.ds(0, c)], sem
        ).wait()
        pltpu.sync_copy(
            rows_v.at[pl.ds(0, c)], out_hbm.at[pl.ds(base + off, c)])
        off += c


def kernel(x, y, W_lookup):
    del x
    return _sc_gather(y, W_lookup)


# speedup vs baseline: 1.0272x; 1.0272x over previous
import functools

import jax
import jax.numpy as jnp
from jax import lax
from jax.experimental import pallas as pl
from jax.experimental.pallas import tpu as pltpu
from jax.experimental.pallas import tpu_sc as plsc

_NUM_CORES = 2
_NUM_SUBCORES = 16
_NUM_WORKERS = _NUM_CORES * _NUM_SUBCORES

_BATCH = 16384
_D_MODEL = 768
_ROWS_PER_WORKER = _BATCH // _NUM_WORKERS
_BUF_ROWS = 80
_CHUNKS = (80, 80, 80, 80, 80, 80, 32)
_OFFS = tuple(sum(_CHUNKS[:i]) for i in range(len(_CHUNKS)))


@functools.partial(
    pl.kernel,
    out_type=jax.ShapeDtypeStruct((_BATCH, _D_MODEL), jnp.float32),
    mesh=plsc.VectorSubcoreMesh(core_axis_name="c", subcore_axis_name="s"),
    scratch_types=[
        pltpu.VMEM((_ROWS_PER_WORKER,), jnp.int32),
        pltpu.VMEM((_BUF_ROWS, _D_MODEL), jnp.float32),
        pltpu.VMEM((_BUF_ROWS, _D_MODEL), jnp.float32),
        pltpu.SemaphoreType.DMA,
        pltpu.SemaphoreType.DMA,
        pltpu.SemaphoreType.DMA,
        pltpu.SemaphoreType.DMA,
    ],
)
def _sc_gather(idx_hbm, table_hbm, out_hbm, idx_v, rows0, rows1,
               g0, g1, o0, o1):
    wid = lax.axis_index("s") * _NUM_CORES + lax.axis_index("c")
    base = wid * _ROWS_PER_WORKER
    pltpu.sync_copy(idx_hbm.at[pl.ds(base, _ROWS_PER_WORKER)], idx_v)

    bufs = (rows0, rows1)
    gsems = (g0, g1)
    osems = (o0, o1)
    n = len(_CHUNKS)

    def gather(k, buf, sem):
        c = _CHUNKS[k]
        return pltpu.async_copy(
            table_hbm.at[idx_v.at[pl.ds(_OFFS[k], c)]],
            buf.at[pl.ds(0, c)], sem)

    def put(k, buf, sem):
        c = _CHUNKS[k]
        return pltpu.async_copy(
            buf.at[pl.ds(0, c)], out_hbm.at[pl.ds(base + _OFFS[k], c)], sem)

    g_descs = [gather(0, bufs[0], gsems[0]), None]
    out_descs = [None, None]
    for i in range(n):
        b, nb = i % 2, (i + 1) % 2
        if i + 1 < n:
            if out_descs[nb] is not None:
                out_descs[nb].wait()
            g_descs[nb] = gather(i + 1, bufs[nb], gsems[nb])
        g_descs[b].wait()
        out_descs[b] = put(i, bufs[b], osems[b])
    out_descs[(n - 1) % 2].wait()
    out_descs[(n - 2) % 2].wait()


def kernel(x, y, W_lookup):
    del x
    return _sc_gather(y, W_lookup)
